# baseline (speedup 1.0000x reference)
import functools

import jax
import jax.numpy as jnp
from jax import lax
from jax.experimental import pallas as pl
from jax.experimental.pallas import tpu as pltpu
from jax.experimental.pallas import tpu_sc as plsc

_N = 10000
_M = 10000
_P = 320000
_C = 128
_K = 64
_DEPTH = 2
_HEADS = 4
_NEG = 0.2

_NC = 2
_NS = 16
_NW = _NC * _NS
_CHUNK = 80
_PPT = _P // _NW
_NCHUNK = _PPT // _CHUNK
_RPT = _M // _NS

_HIGH = lax.Precision.HIGHEST

_SC_PARAMS = pltpu.CompilerParams(use_tc_tiling_on_sc=False,
                                  needs_layout_passes=False)

_mesh = plsc.VectorSubcoreMesh(core_axis_name="c", subcore_axis_name="s",
                               num_cores=_NC, num_subcores=_NS)


def _lin_body(cout, r, h_ref, w_ref, b_ref, ad_ref, xlp_ref, xd_ref):
    xl = jnp.dot(h_ref[...], w_ref[...], precision=_HIGH,
                 preferred_element_type=jnp.float32) + b_ref[...]
    xlp_ref[:, :cout] = xl
    pad = jnp.where(
        lax.broadcasted_iota(jnp.int32, (xl.shape[0], r - cout), 1) == 0,
        1.0, 0.0)
    xlp_ref[:, cout:] = pad
    xd_ref[...] = jnp.dot(xl, ad_ref[...], precision=_HIGH,
                          preferred_element_type=jnp.float32)


def _tc_linear(h, w, b, ad, cout, r):
    n, cin = h.shape
    bn = 400
    grid = (n // bn,)
    return pl.pallas_call(
        functools.partial(_lin_body, cout, r),
        grid=grid,
        in_specs=[
            pl.BlockSpec((bn, cin), lambda i: (i, 0)),
            pl.BlockSpec((cin, cout), lambda i: (0, 0)),
            pl.BlockSpec((1, cout), lambda i: (0, 0)),
            pl.BlockSpec((cout, 1), lambda i: (0, 0)),
        ],
        out_specs=[
            pl.BlockSpec((bn, r), lambda i: (i, 0)),
            pl.BlockSpec((bn, 1), lambda i: (i, 0)),
        ],
        out_shape=[
            jax.ShapeDtypeStruct((n, r), jnp.float32),
            jax.ShapeDtypeStruct((n, 1), jnp.float32),
        ],
    )(h, w.astype(jnp.float32), b.reshape(1, cout).astype(jnp.float32),
      ad.reshape(cout, 1).astype(jnp.float32))


def _edge_body(cout, r, yp_ref, ae_ref, ypad_ref, al_ref):
    s = yp_ref[0] + yp_ref[1]
    deg = s[:, cout:cout + 1]
    inv = 1.0 / jnp.maximum(deg, 1.0)
    y = s[:, :cout] * inv
    ypad_ref[:, :cout] = y
    pad = jnp.where(
        lax.broadcasted_iota(jnp.int32, (y.shape[0], r - cout), 1) == 0,
        1.0, 0.0)
    ypad_ref[:, cout:] = pad
    al_ref[...] = jnp.dot(y, ae_ref[...], precision=_HIGH,
                          preferred_element_type=jnp.float32)


def _tc_edge(yparts, ae, cout, r):
    bm = 400
    yp = yparts.reshape(2, _M, r)
    return pl.pallas_call(
        functools.partial(_edge_body, cout, r),
        grid=(_M // bm,),
        in_specs=[
            pl.BlockSpec((2, bm, r), lambda i: (0, i, 0)),
            pl.BlockSpec((cout, 1), lambda i: (0, 0)),
        ],
        out_specs=[
            pl.BlockSpec((bm, r), lambda i: (i, 0)),
            pl.BlockSpec((bm, 1), lambda i: (i, 0)),
        ],
        out_shape=[
            jax.ShapeDtypeStruct((_M, r), jnp.float32),
            jax.ShapeDtypeStruct((_M, 1), jnp.float32),
        ],
    )(yp, ae.reshape(cout, 1).astype(jnp.float32))


def _vert_body(cout, r, nh, *refs):
    out_ref = refs[-1]
    acc = None
    for xr in refs[:-1]:
        s = xr[0] + xr[1]
        den = s[:, cout:cout + 1] + 1e-12
        xo = s[:, :cout] / den
        e = jnp.where(xo > 0, xo, jnp.exp(jnp.minimum(xo, 0.0)) - 1.0)
        acc = e if acc is None else acc + e
    out_ref[...] = acc * (1.0 / nh)


def _tc_vertex(xoparts, cout, r):
    nh = len(xoparts)
    bn = 400
    xps = [x.reshape(2, _N, r) for x in xoparts]
    return pl.pallas_call(
        functools.partial(_vert_body, cout, r, nh),
        grid=(_N // bn,),
        in_specs=[pl.BlockSpec((2, bn, r), lambda i: (0, i, 0))
                  for _ in range(nh)],
        out_specs=pl.BlockSpec((bn, cout), lambda i: (i, 0)),
        out_shape=jax.ShapeDtypeStruct((_N, cout), jnp.float32),
    )(*xps)


def _zero_rowbuf(row_buf, nrow, ncg):
    @pl.loop(0, nrow)
    def _(i):
        for g in range(ncg):
            row_buf[i, pl.ds(g * 16, 16)] = jnp.zeros((16,), jnp.float32)


_NZCHUNK = _M // _CHUNK
_ZPT = -(-_NZCHUNK // _NS)


def _zero_and_barrier(row_buf, acc_sh, s, nrow, ncg):
    _zero_rowbuf(row_buf, nrow, ncg)

    @pl.loop(0, _ZPT)
    def _(k):
        j = s + k * _NS

        @pl.when(j < _NZCHUNK)
        def _():
            pltpu.sync_copy(row_buf, acc_sh.at[pl.ds(j * nrow, nrow)])

    plsc.subcore_barrier()


def _writeout(acc_sh, out_hbm, c, s, nrow, r):
    plsc.subcore_barrier()

    @pl.loop(0, _ZPT)
    def _(k):
        j = s + k * _NS

        @pl.when(j < _NZCHUNK)
        def _():
            off = j * nrow
            pltpu.sync_copy(acc_sh.at[pl.ds(off, nrow)],
                            out_hbm.at[pl.ds(c * _M + off, nrow)])


def _v2e_body(r, xlp_hbm, pv_hbm, pe_hbm, out_hbm,
              pv_buf, pe_buf, row_buf, acc_sh, sem):
    c = lax.axis_index("c")
    s = lax.axis_index("s")
    ncg = r // 16
    _zero_and_barrier(row_buf, acc_sh, s, _CHUNK, ncg)

    @pl.loop(0, _NCHUNK)
    def _(k):
        base = c * (_P // _NC) + s * _PPT + k * _CHUNK
        pltpu.sync_copy(pv_hbm.at[pl.ds(base, _CHUNK)], pv_buf)
        pltpu.sync_copy(pe_hbm.at[pl.ds(base, _CHUNK)], pe_buf)
        pltpu.async_copy(xlp_hbm.at[pv_buf], row_buf, sem).wait()
        pltpu.sync_copy(row_buf, acc_sh.at[pe_buf], add=True)

    _writeout(acc_sh, out_hbm, c, s, _CHUNK, r)


def _sc_v2e(xl_pad, pv, pe, r):
    k = pl.kernel(
        functools.partial(_v2e_body, r),
        out_type=jax.ShapeDtypeStruct((_NC * _M, r), jnp.float32),
        mesh=_mesh,
        scratch_types=[
            pltpu.VMEM((_CHUNK,), jnp.int32),
            pltpu.VMEM((_CHUNK,), jnp.int32),
            pltpu.VMEM((_CHUNK, r), jnp.float32),
            pltpu.VMEM_SHARED((_M, r), jnp.float32),
            pltpu.SemaphoreType.DMA,
        ],
        compiler_params=_SC_PARAMS,
    )
    return k(xl_pad, pv, pe)


def _e2v_body(r, ypad_hbm, al_hbm, xd_hbm, pv_hbm, pe_hbm, out_hbm,
              pv_buf, pe_buf, row_buf, ex_buf, al_buf, xd_buf, acc_sh, sem):
    c = lax.axis_index("c")
    s = lax.axis_index("s")
    ncg = r // 16
    pltpu.sync_copy(al_hbm, al_buf)
    pltpu.sync_copy(xd_hbm, xd_buf)
    _zero_and_barrier(row_buf, acc_sh, s, _CHUNK, ncg)

    @pl.loop(0, _NCHUNK)
    def _(k):
        base = c * (_P // _NC) + s * _PPT + k * _CHUNK
        pltpu.sync_copy(pv_hbm.at[pl.ds(base, _CHUNK)], pv_buf)
        pltpu.sync_copy(pe_hbm.at[pl.ds(base, _CHUNK)], pe_buf)
        pltpu.async_copy(ypad_hbm.at[pe_buf], row_buf, sem).wait()

        @pl.loop(0, _CHUNK // 16)
        def _(g):
            pvv = pv_buf[pl.ds(g * 16, 16)]
            pee = pe_buf[pl.ds(g * 16, 16)]
            sc = plsc.load_gather(al_buf, [pee]) + plsc.load_gather(xd_buf, [pvv])
            sc = jnp.where(sc >= 0, sc, _NEG * sc)
            ex_buf[pl.ds(g * 16, 16)] = jnp.exp(sc)

        @pl.loop(0, _CHUNK, unroll=4)
        def _(i):
            exi = plsc.load_gather(ex_buf, [jnp.zeros((16,), jnp.int32) + i])
            for g in range(ncg):
                row_buf[i, pl.ds(g * 16, 16)] = (
                    row_buf[i, pl.ds(g * 16, 16)] * exi)

        pltpu.sync_copy(row_buf, acc_sh.at[pv_buf], add=True)

    _writeout(acc_sh, out_hbm, c, s, _CHUNK, r)


def _sc_e2v(y_pad, alpha, xdst, pv, pe, r):
    k = pl.kernel(
        functools.partial(_e2v_body, r),
        out_type=jax.ShapeDtypeStruct((_NC * _N, r), jnp.float32),
        mesh=_mesh,
        scratch_types=[
            pltpu.VMEM((_CHUNK,), jnp.int32),
            pltpu.VMEM((_CHUNK,), jnp.int32),
            pltpu.VMEM((_CHUNK, r), jnp.float32),
            pltpu.VMEM((_CHUNK,), jnp.float32),
            pltpu.VMEM((_M,), jnp.float32),
            pltpu.VMEM((_N,), jnp.float32),
            pltpu.VMEM_SHARED((_N, r), jnp.float32),
            pltpu.SemaphoreType.DMA,
        ],
        compiler_params=_SC_PARAMS,
    )
    return k(y_pad, alpha, xdst, pv, pe)


def _conv(h, pv, pe, w, b, ae, ad, cout):
    r = cout + 16
    xl_pad, xdst = _tc_linear(h, w, b, ad, cout, r)
    yparts = _sc_v2e(xl_pad, pv, pe, r)
    y_pad, alpha = _tc_edge(yparts, ae, cout, r)
    xoparts = _sc_e2v(y_pad, alpha.reshape(_M), xdst.reshape(_N), pv, pe, r)
    return xoparts


def kernel(X, pair_v, pair_e, W_layers, b_layers, ae_layers, ad_layers,
           W_out, b_out, ae_out, ad_out):
    h = X.astype(jnp.float32)
    pv = pair_v.astype(jnp.int32)
    pe = pair_e.astype(jnp.int32)
    for l in range(_DEPTH):
        parts = [
            _conv(h, pv, pe, W_layers[l, hd], b_layers[l, hd],
                  ae_layers[l, hd], ad_layers[l, hd], _C)
            for hd in range(_HEADS)
        ]
        h = _tc_vertex(parts, _C, _C + 16)
    xop = _conv(h, pv, pe, W_out, b_out, ae_out, ad_out, _K)
    return _tc_vertex([xop], _K, _K + 16)


# speedup vs baseline: 1.6479x; 1.6479x over previous
import functools

import jax
import jax.numpy as jnp
from jax import lax
from jax.experimental import pallas as pl
from jax.experimental.pallas import tpu as pltpu
from jax.experimental.pallas import tpu_sc as plsc

_N = 10000
_M = 10000
_P = 320000
_C = 128
_K = 64
_DEPTH = 2
_HEADS = 4
_NEG = 0.2

_NC = 2
_NS = 16
_NW = _NC * _NS
_CHUNK = 80
_PPT = _P // _NW
_NCHUNK = _PPT // _CHUNK
_RPT = _M // _NS

_HIGH = lax.Precision.HIGHEST

_SC_PARAMS = pltpu.CompilerParams(use_tc_tiling_on_sc=False,
                                  needs_layout_passes=False)

_mesh = plsc.VectorSubcoreMesh(core_axis_name="c", subcore_axis_name="s",
                               num_cores=_NC, num_subcores=_NS)


def _lin_body(cout, r, h_ref, w_ref, b_ref, ad_ref, xlp_ref, xd_ref):
    xl = jnp.dot(h_ref[...], w_ref[...], precision=_HIGH,
                 preferred_element_type=jnp.float32) + b_ref[...]
    xlp_ref[:, :cout] = xl
    pad = jnp.where(
        lax.broadcasted_iota(jnp.int32, (xl.shape[0], r - cout), 1) == 0,
        1.0, 0.0)
    xlp_ref[:, cout:] = pad
    xd_ref[...] = jnp.dot(xl, ad_ref[...], precision=_HIGH,
                          preferred_element_type=jnp.float32)


def _tc_linear(h, w, b, ad, cout, r):
    n, cin = h.shape
    bn = 400
    grid = (n // bn,)
    return pl.pallas_call(
        functools.partial(_lin_body, cout, r),
        grid=grid,
        in_specs=[
            pl.BlockSpec((bn, cin), lambda i: (i, 0)),
            pl.BlockSpec((cin, cout), lambda i: (0, 0)),
            pl.BlockSpec((1, cout), lambda i: (0, 0)),
            pl.BlockSpec((cout, 1), lambda i: (0, 0)),
        ],
        out_specs=[
            pl.BlockSpec((bn, r), lambda i: (i, 0)),
            pl.BlockSpec((bn, 1), lambda i: (i, 0)),
        ],
        out_shape=[
            jax.ShapeDtypeStruct((n, r), jnp.float32),
            jax.ShapeDtypeStruct((n, 1), jnp.float32),
        ],
    )(h, w.astype(jnp.float32), b.reshape(1, cout).astype(jnp.float32),
      ad.reshape(cout, 1).astype(jnp.float32))


def _edge_body(cout, r, yp_ref, ae_ref, ypad_ref):
    s = yp_ref[0] + yp_ref[1]
    deg = s[:, cout:cout + 1]
    inv = 1.0 / jnp.maximum(deg, 1.0)
    y = s[:, :cout] * inv
    ypad_ref[:, :cout] = y
    alpha = jnp.dot(y, ae_ref[...], precision=_HIGH,
                    preferred_element_type=jnp.float32)
    lane = lax.broadcasted_iota(jnp.int32, (y.shape[0], r - cout), 1)
    pad = jnp.where(lane == 0, 1.0, jnp.where(lane == 1, alpha, 0.0))
    ypad_ref[:, cout:] = pad


def _tc_edge(yparts, ae, cout, r):
    bm = 400
    yp = yparts.reshape(2, _M, r)
    return pl.pallas_call(
        functools.partial(_edge_body, cout, r),
        grid=(_M // bm,),
        in_specs=[
            pl.BlockSpec((2, bm, r), lambda i: (0, i, 0)),
            pl.BlockSpec((cout, 1), lambda i: (0, 0)),
        ],
        out_specs=pl.BlockSpec((bm, r), lambda i: (i, 0)),
        out_shape=jax.ShapeDtypeStruct((_M, r), jnp.float32),
    )(yp, ae.reshape(cout, 1).astype(jnp.float32))


def _vert_body(cout, r, nh, *refs):
    out_ref = refs[-1]
    acc = None
    for xr in refs[:-1]:
        s = xr[0] + xr[1]
        den = s[:, cout:cout + 1] + 1e-12
        xo = s[:, :cout] / den
        e = jnp.where(xo > 0, xo, jnp.exp(jnp.minimum(xo, 0.0)) - 1.0)
        acc = e if acc is None else acc + e
    out_ref[...] = acc * (1.0 / nh)


def _tc_vertex(xoparts, cout, r):
    nh = len(xoparts)
    bn = 400
    xps = [x.reshape(2, _N, r) for x in xoparts]
    return pl.pallas_call(
        functools.partial(_vert_body, cout, r, nh),
        grid=(_N // bn,),
        in_specs=[pl.BlockSpec((2, bn, r), lambda i: (0, i, 0))
                  for _ in range(nh)],
        out_specs=pl.BlockSpec((bn, cout), lambda i: (i, 0)),
        out_shape=jax.ShapeDtypeStruct((_N, cout), jnp.float32),
    )(*xps)


def _zero_rowbuf(row_buf, nrow, ncg):
    @pl.loop(0, nrow)
    def _(i):
        for g in range(ncg):
            row_buf[i, pl.ds(g * 16, 16)] = jnp.zeros((16,), jnp.float32)


_NZCHUNK = _M // _CHUNK
_ZPT = -(-_NZCHUNK // _NS)


def _zero_and_barrier(row_buf, acc_sh, s, nrow, ncg):
    _zero_rowbuf(row_buf, nrow, ncg)

    @pl.loop(0, _ZPT)
    def _(k):
        j = s + k * _NS

        @pl.when(j < _NZCHUNK)
        def _():
            pltpu.sync_copy(row_buf, acc_sh.at[pl.ds(j * nrow, nrow)])

    plsc.subcore_barrier()


def _writeout(acc_sh, out_hbm, c, s, nrow, r):
    plsc.subcore_barrier()

    @pl.loop(0, _ZPT)
    def _(k):
        j = s + k * _NS

        @pl.when(j < _NZCHUNK)
        def _():
            off = j * nrow
            pltpu.sync_copy(acc_sh.at[pl.ds(off, nrow)],
                            out_hbm.at[pl.ds(c * _M + off, nrow)])


_GV = 3
_GE = 2


def _v2e_group(k0, nb, wid, xlp_hbm, pv3_hbm, pe3_hbm,
               pv_buf, pe_buf, row_bufs, acc_sh, sem_i, sem_g, sem_s):
    di = [pltpu.async_copy(pv3_hbm.at[wid, pl.ds(k0, nb)],
                           pv_buf.at[pl.ds(0, nb)], sem_i),
          pltpu.async_copy(pe3_hbm.at[wid, pl.ds(k0, nb)],
                           pe_buf.at[pl.ds(0, nb)], sem_i)]
    for d in di:
        d.wait()
    gd = [pltpu.async_copy(xlp_hbm.at[pv_buf.at[b]], row_bufs.at[b], sem_g)
          for b in range(nb)]
    for d in gd:
        d.wait()
    sd = [pltpu.async_copy(row_bufs.at[b], acc_sh.at[pe_buf.at[b]],
                           sem_s, add=True) for b in range(nb)]
    for d in sd:
        d.wait()


def _v2e_body(r, xlp_hbm, pv3_hbm, pe3_hbm, out_hbm,
              pv_buf, pe_buf, row_bufs, acc_sh, sem_i, sem_g, sem_s):
    c = lax.axis_index("c")
    s = lax.axis_index("s")
    wid = c * _NS + s
    ncg = r // 16
    _zero_and_barrier(row_bufs.at[0], acc_sh, s, _CHUNK, ncg)

    nfull = _NCHUNK // _GV
    tail = _NCHUNK % _GV

    @pl.loop(0, nfull)
    def _(g):
        _v2e_group(g * _GV, _GV, wid, xlp_hbm, pv3_hbm, pe3_hbm,
                   pv_buf, pe_buf, row_bufs, acc_sh, sem_i, sem_g, sem_s)

    if tail:
        _v2e_group(nfull * _GV, tail, wid, xlp_hbm, pv3_hbm, pe3_hbm,
                   pv_buf, pe_buf, row_bufs, acc_sh, sem_i, sem_g, sem_s)

    _writeout(acc_sh, out_hbm, c, s, _CHUNK, r)


def _sc_v2e(xl_pad, pv3, pe3, r):
    k = pl.kernel(
        functools.partial(_v2e_body, r),
        out_type=jax.ShapeDtypeStruct((_NC * _M, r), jnp.float32),
        mesh=_mesh,
        scratch_types=[
            pltpu.VMEM((_GV, _CHUNK), jnp.int32),
            pltpu.VMEM((_GV, _CHUNK), jnp.int32),
            pltpu.VMEM((_GV, _CHUNK, r), jnp.float32),
            pltpu.VMEM_SHARED((_M, r), jnp.float32),
            pltpu.SemaphoreType.DMA,
            pltpu.SemaphoreType.DMA,
            pltpu.SemaphoreType.DMA,
        ],
        compiler_params=_SC_PARAMS,
    )
    return k(xl_pad, pv3, pe3)


def _e2v_group(k0, nb, cout, ncg, wid, ypad_hbm, pv3_hbm, pe3_hbm,
               pv_buf, pe_buf, row_bufs, ex_buf, xd_buf, acc_sh,
               sem_i, sem_g, sem_s):
    di = [pltpu.async_copy(pv3_hbm.at[wid, pl.ds(k0, nb)],
                           pv_buf.at[pl.ds(0, nb)], sem_i),
          pltpu.async_copy(pe3_hbm.at[wid, pl.ds(k0, nb)],
                           pe_buf.at[pl.ds(0, nb)], sem_i)]
    for d in di:
        d.wait()
    gd = [pltpu.async_copy(ypad_hbm.at[pe_buf.at[b]], row_bufs.at[b], sem_g)
          for b in range(nb)]
    sd = []
    for b in range(nb):
        gd[b].wait()

        @pl.loop(0, _CHUNK // 16)
        def _(g, b=b):
            g16 = g * 16 + lax.iota(jnp.int32, 16)
            al = plsc.load_gather(row_bufs.at[b],
                                  [g16, jnp.full((16,), cout + 1, jnp.int32)])
            pvv = pv_buf[b, pl.ds(g * 16, 16)]
            xd = plsc.load_gather(xd_buf, [pvv])
            sc = al + xd
            sc = jnp.where(sc >= 0, sc, _NEG * sc)
            ex_buf[pl.ds(g * 16, 16)] = jnp.exp(sc)

        @pl.loop(0, _CHUNK, unroll=4)
        def _(i, b=b):
            exi = plsc.load_gather(ex_buf, [jnp.zeros((16,), jnp.int32) + i])
            for g in range(ncg):
                row_bufs[b, i, pl.ds(g * 16, 16)] = (
                    row_bufs[b, i, pl.ds(g * 16, 16)] * exi)

        sd.append(pltpu.async_copy(row_bufs.at[b], acc_sh.at[pv_buf.at[b]],
                                   sem_s, add=True))
    for d in sd:
        d.wait()


def _e2v_body(cout, r, ypad_hbm, xd_hbm, pv3_hbm, pe3_hbm, out_hbm,
              pv_buf, pe_buf, row_bufs, ex_buf, xd_buf, acc_sh,
              sem_i, sem_g, sem_s):
    c = lax.axis_index("c")
    s = lax.axis_index("s")
    wid = c * _NS + s
    ncg = r // 16
    pltpu.sync_copy(xd_hbm, xd_buf)
    _zero_and_barrier(row_bufs.at[0], acc_sh, s, _CHUNK, ncg)

    nfull = _NCHUNK // _GE
    tail = _NCHUNK % _GE

    @pl.loop(0, nfull)
    def _(g):
        _e2v_group(g * _GE, _GE, cout, ncg, wid, ypad_hbm, pv3_hbm, pe3_hbm,
                   pv_buf, pe_buf, row_bufs, ex_buf, xd_buf, acc_sh,
                   sem_i, sem_g, sem_s)

    if tail:
        _e2v_group(nfull * _GE, tail, cout, ncg, wid, ypad_hbm, pv3_hbm,
                   pe3_hbm, pv_buf, pe_buf, row_bufs, ex_buf, xd_buf, acc_sh,
                   sem_i, sem_g, sem_s)

    _writeout(acc_sh, out_hbm, c, s, _CHUNK, r)


def _sc_e2v(y_pad, xdst, pv3, pe3, cout, r):
    k = pl.kernel(
        functools.partial(_e2v_body, cout, r),
        out_type=jax.ShapeDtypeStruct((_NC * _N, r), jnp.float32),
        mesh=_mesh,
        scratch_types=[
            pltpu.VMEM((_GE, _CHUNK), jnp.int32),
            pltpu.VMEM((_GE, _CHUNK), jnp.int32),
            pltpu.VMEM((_GE, _CHUNK, r), jnp.float32),
            pltpu.VMEM((_CHUNK,), jnp.float32),
            pltpu.VMEM((_N,), jnp.float32),
            pltpu.VMEM_SHARED((_N, r), jnp.float32),
            pltpu.SemaphoreType.DMA,
            pltpu.SemaphoreType.DMA,
            pltpu.SemaphoreType.DMA,
        ],
        compiler_params=_SC_PARAMS,
    )
    return k(y_pad, xdst, pv3, pe3)


def _conv(h, pv3, pe3, w, b, ae, ad, cout):
    r = cout + 16
    xl_pad, xdst = _tc_linear(h, w, b, ad, cout, r)
    yparts = _sc_v2e(xl_pad, pv3, pe3, r)
    y_pad = _tc_edge(yparts, ae, cout, r)
    xoparts = _sc_e2v(y_pad, xdst.reshape(_N), pv3, pe3, cout, r)
    return xoparts


def kernel(X, pair_v, pair_e, W_layers, b_layers, ae_layers, ad_layers,
           W_out, b_out, ae_out, ad_out):
    h = X.astype(jnp.float32)
    pv = pair_v.astype(jnp.int32).reshape(_NW, _NCHUNK, _CHUNK)
    pe = pair_e.astype(jnp.int32).reshape(_NW, _NCHUNK, _CHUNK)
    for l in range(_DEPTH):
        parts = [
            _conv(h, pv, pe, W_layers[l, hd], b_layers[l, hd],
                  ae_layers[l, hd], ad_layers[l, hd], _C)
            for hd in range(_HEADS)
        ]
        h = _tc_vertex(parts, _C, _C + 16)
    xop = _conv(h, pv, pe, W_out, b_out, ae_out, ad_out, _K)
    return _tc_vertex([xop], _K, _K + 16)


# speedup vs baseline: 2.0684x; 1.2552x over previous
import functools

import jax
import jax.numpy as jnp
from jax import lax
from jax.experimental import pallas as pl
from jax.experimental.pallas import tpu as pltpu
from jax.experimental.pallas import tpu_sc as plsc

_N = 10000
_M = 10000
_P = 320000
_C = 128
_K = 64
_DEPTH = 2
_HEADS = 4
_NEG = 0.2

_NC = 2
_NS = 16
_NW = _NC * _NS
_CHUNK = 80
_PPT = _P // _NW
_NCHUNK = _PPT // _CHUNK
_RPT = _M // _NS

_HIGH = lax.Precision.HIGHEST

_SC_PARAMS = pltpu.CompilerParams(use_tc_tiling_on_sc=False,
                                  needs_layout_passes=False)

_mesh = plsc.VectorSubcoreMesh(core_axis_name="c", subcore_axis_name="s",
                               num_cores=_NC, num_subcores=_NS)


def _lin_body(cout, r, h_ref, w_ref, b_ref, ad_ref, xlp_ref, xd_ref):
    xl = jnp.dot(h_ref[...], w_ref[...], precision=_HIGH,
                 preferred_element_type=jnp.float32) + b_ref[...]
    xlp_ref[:, :cout] = xl
    pad = jnp.where(
        lax.broadcasted_iota(jnp.int32, (xl.shape[0], r - cout), 1) == 0,
        1.0, 0.0)
    xlp_ref[:, cout:] = pad
    xd_ref[...] = jnp.dot(xl, ad_ref[...], precision=_HIGH,
                          preferred_element_type=jnp.float32)


def _tc_linear(h, w, b, ad, cout, r):
    n, cin = h.shape
    bn = 400
    grid = (n // bn,)
    return pl.pallas_call(
        functools.partial(_lin_body, cout, r),
        grid=grid,
        in_specs=[
            pl.BlockSpec((bn, cin), lambda i: (i, 0)),
            pl.BlockSpec((cin, cout), lambda i: (0, 0)),
            pl.BlockSpec((1, cout), lambda i: (0, 0)),
            pl.BlockSpec((cout, 1), lambda i: (0, 0)),
        ],
        out_specs=[
            pl.BlockSpec((bn, r), lambda i: (i, 0)),
            pl.BlockSpec((bn, 1), lambda i: (i, 0)),
        ],
        out_shape=[
            jax.ShapeDtypeStruct((n, r), jnp.float32),
            jax.ShapeDtypeStruct((n, 1), jnp.float32),
        ],
    )(h, w.astype(jnp.float32), b.reshape(1, cout).astype(jnp.float32),
      ad.reshape(cout, 1).astype(jnp.float32))


def _edge_body(cout, r, yp_ref, ae_ref, ypad_ref):
    s = yp_ref[0] + yp_ref[1]
    deg = s[:, cout:cout + 1]
    inv = 1.0 / jnp.maximum(deg, 1.0)
    y = s[:, :cout] * inv
    ypad_ref[:, :cout] = y
    alpha = jnp.dot(y, ae_ref[...], precision=_HIGH,
                    preferred_element_type=jnp.float32)
    lane = lax.broadcasted_iota(jnp.int32, (y.shape[0], r - cout), 1)
    pad = jnp.where(lane == 0, 1.0, jnp.where(lane == 1, alpha, 0.0))
    ypad_ref[:, cout:] = pad


def _tc_edge(yparts, ae, cout, r):
    bm = 400
    yp = yparts.reshape(2, _M, r)
    return pl.pallas_call(
        functools.partial(_edge_body, cout, r),
        grid=(_M // bm,),
        in_specs=[
            pl.BlockSpec((2, bm, r), lambda i: (0, i, 0)),
            pl.BlockSpec((cout, 1), lambda i: (0, 0)),
        ],
        out_specs=pl.BlockSpec((bm, r), lambda i: (i, 0)),
        out_shape=jax.ShapeDtypeStruct((_M, r), jnp.float32),
    )(yp, ae.reshape(cout, 1).astype(jnp.float32))


def _vert_body(cout, r, nh, *refs):
    out_ref = refs[-1]
    acc = None
    for xr in refs[:-1]:
        s = xr[0] + xr[1]
        den = s[:, cout:cout + 1] + 1e-12
        xo = s[:, :cout] / den
        e = jnp.where(xo > 0, xo, jnp.exp(jnp.minimum(xo, 0.0)) - 1.0)
        acc = e if acc is None else acc + e
    out_ref[...] = acc * (1.0 / nh)


def _tc_vertex(xoparts, cout, r):
    nh = len(xoparts)
    bn = 400
    xps = [x.reshape(2, _N, r) for x in xoparts]
    return pl.pallas_call(
        functools.partial(_vert_body, cout, r, nh),
        grid=(_N // bn,),
        in_specs=[pl.BlockSpec((2, bn, r), lambda i: (0, i, 0))
                  for _ in range(nh)],
        out_specs=pl.BlockSpec((bn, cout), lambda i: (i, 0)),
        out_shape=jax.ShapeDtypeStruct((_N, cout), jnp.float32),
    )(*xps)


def _zero_rowbuf(row_buf, nrow, ncg):
    @pl.loop(0, nrow)
    def _(i):
        for g in range(ncg):
            row_buf[i, pl.ds(g * 16, 16)] = jnp.zeros((16,), jnp.float32)


_NZCHUNK = _M // _CHUNK
_ZPT = -(-_NZCHUNK // _NS)


def _zero_and_barrier(row_buf, acc_sh, s, nrow, ncg):
    _zero_rowbuf(row_buf, nrow, ncg)

    @pl.loop(0, _ZPT)
    def _(k):
        j = s + k * _NS

        @pl.when(j < _NZCHUNK)
        def _():
            pltpu.sync_copy(row_buf, acc_sh.at[pl.ds(j * nrow, nrow)])

    plsc.subcore_barrier()


def _writeout(acc_sh, out_hbm, c, s, nrow, r):
    plsc.subcore_barrier()

    @pl.loop(0, _ZPT)
    def _(k):
        j = s + k * _NS

        @pl.when(j < _NZCHUNK)
        def _():
            off = j * nrow
            pltpu.sync_copy(acc_sh.at[pl.ds(off, nrow)],
                            out_hbm.at[pl.ds(c * _M + off, nrow)])


_SCH = 15
_NFULL = _NCHUNK // _SCH
_TAIL = _NCHUNK % _SCH


def _drain_all(nbuf, row_bufs, acc_sh, idx_buf, sems):
    for b in range(nbuf):
        pltpu.make_async_copy(row_bufs.at[b], acc_sh.at[idx_buf.at[0, 1]],
                              sems[b]).wait()


def _v2e_steps(nsteps, xlp_hbm, idx_buf, row_bufs, acc_sh, sems, sem_g):
    gd = {}
    for i in range(nsteps):
        b = i % 3
        if i >= 3:
            pltpu.make_async_copy(row_bufs.at[b],
                                  acc_sh.at[idx_buf.at[i - 3, 1]],
                                  sems[b]).wait()
        gd[i] = pltpu.async_copy(xlp_hbm.at[idx_buf.at[i, 0]],
                                 row_bufs.at[b], sem_g)
        if i >= 1:
            gd[i - 1].wait()
            pltpu.async_copy(row_bufs.at[(i - 1) % 3],
                             acc_sh.at[idx_buf.at[i - 1, 1]],
                             sems[(i - 1) % 3], add=True)
    gd[nsteps - 1].wait()
    pltpu.async_copy(row_bufs.at[(nsteps - 1) % 3],
                     acc_sh.at[idx_buf.at[nsteps - 1, 1]],
                     sems[(nsteps - 1) % 3], add=True)


def _v2e_body(r, xlp_hbm, pvpe_hbm, out_hbm,
              idx_buf, row_bufs, acc_sh, sem_i, sem_g, s0, s1, s2):
    c = lax.axis_index("c")
    s = lax.axis_index("s")
    wid = c * _NS + s
    sems = (s0, s1, s2)
    _zero_and_barrier(row_bufs.at[0], acc_sh, s, _CHUNK, r // 16)

    @pl.loop(0, _NFULL)
    def _(j):
        @pl.when(j > 0)
        def _():
            _drain_all(3, row_bufs, acc_sh, idx_buf, sems)

        pltpu.async_copy(pvpe_hbm.at[wid, pl.ds(j * _SCH, _SCH)],
                         idx_buf, sem_i).wait()
        _v2e_steps(_SCH, xlp_hbm, idx_buf, row_bufs, acc_sh, sems, sem_g)

    if _TAIL:
        _drain_all(3, row_bufs, acc_sh, idx_buf, sems)
        pltpu.async_copy(pvpe_hbm.at[wid, pl.ds(_NFULL * _SCH, _TAIL)],
                         idx_buf.at[pl.ds(0, _TAIL)], sem_i).wait()
        _v2e_steps(_TAIL, xlp_hbm, idx_buf, row_bufs, acc_sh, sems, sem_g)

    _drain_all(min(3, _TAIL if _TAIL else _SCH), row_bufs, acc_sh,
               idx_buf, sems)
    _writeout(acc_sh, out_hbm, c, s, _CHUNK, r)


def _sc_v2e(xl_pad, pvpe, r):
    k = pl.kernel(
        functools.partial(_v2e_body, r),
        out_type=jax.ShapeDtypeStruct((_NC * _M, r), jnp.float32),
        mesh=_mesh,
        scratch_types=[
            pltpu.VMEM((_SCH, 2, _CHUNK), jnp.int32),
            pltpu.VMEM((3, _CHUNK, r), jnp.float32),
            pltpu.VMEM_SHARED((_M, r), jnp.float32),
            pltpu.SemaphoreType.DMA,
            pltpu.SemaphoreType.DMA,
            pltpu.SemaphoreType.DMA,
            pltpu.SemaphoreType.DMA,
            pltpu.SemaphoreType.DMA,
        ],
        compiler_params=_SC_PARAMS,
    )
    return k(xl_pad, pvpe)


def _e2v_compute(b, cout, ncg, idx_buf, i, row_bufs, ex_buf, xd_buf):
    @pl.loop(0, _CHUNK // 16)
    def _(g):
        g16 = g * 16 + lax.iota(jnp.int32, 16)
        al = plsc.load_gather(row_bufs.at[b],
                              [g16, jnp.full((16,), cout + 1, jnp.int32)])
        pvv = idx_buf[i, 0, pl.ds(g * 16, 16)]
        xd = plsc.load_gather(xd_buf, [pvv])
        sc = al + xd
        sc = jnp.where(sc >= 0, sc, _NEG * sc)
        ex_buf[pl.ds(g * 16, 16)] = jnp.exp(sc)

    @pl.loop(0, _CHUNK, unroll=4)
    def _(rr):
        exi = plsc.load_gather(ex_buf, [jnp.zeros((16,), jnp.int32) + rr])
        for g in range(ncg):
            row_bufs[b, rr, pl.ds(g * 16, 16)] = (
                row_bufs[b, rr, pl.ds(g * 16, 16)] * exi)


def _e2v_steps(nsteps, cout, ncg, ypad_hbm, idx_buf, row_bufs,
               ex_buf, xd_buf, acc_sh, sems, sem_g):
    gd = {}

    def process(i):
        b = i % 2
        gd[i].wait()
        _e2v_compute(b, cout, ncg, idx_buf, i, row_bufs, ex_buf, xd_buf)
        pltpu.async_copy(row_bufs.at[b], acc_sh.at[idx_buf.at[i, 0]],
                         sems[b], add=True)

    for i in range(nsteps):
        b = i % 2
        if i >= 2:
            pltpu.make_async_copy(row_bufs.at[b],
                                  acc_sh.at[idx_buf.at[i - 2, 0]],
                                  sems[b]).wait()
        gd[i] = pltpu.async_copy(ypad_hbm.at[idx_buf.at[i, 1]],
                                 row_bufs.at[b], sem_g)
        if i >= 1:
            process(i - 1)
    process(nsteps - 1)


def _e2v_body(cout, r, ypad_hbm, xd_hbm, pvpe_hbm, out_hbm,
              idx_buf, row_bufs, ex_buf, xd_buf, acc_sh,
              sem_i, sem_g, s0, s1):
    c = lax.axis_index("c")
    s = lax.axis_index("s")
    wid = c * _NS + s
    ncg = r // 16
    sems = (s0, s1)
    pltpu.sync_copy(xd_hbm, xd_buf)
    _zero_and_barrier(row_bufs.at[0], acc_sh, s, _CHUNK, ncg)

    @pl.loop(0, _NFULL)
    def _(j):
        @pl.when(j > 0)
        def _():
            _drain_all(2, row_bufs, acc_sh, idx_buf, sems)

        pltpu.async_copy(pvpe_hbm.at[wid, pl.ds(j * _SCH, _SCH)],
                         idx_buf, sem_i).wait()
        _e2v_steps(_SCH, cout, ncg, ypad_hbm, idx_buf, row_bufs,
                   ex_buf, xd_buf, acc_sh, sems, sem_g)

    if _TAIL:
        _drain_all(2, row_bufs, acc_sh, idx_buf, sems)
        pltpu.async_copy(pvpe_hbm.at[wid, pl.ds(_NFULL * _SCH, _TAIL)],
                         idx_buf.at[pl.ds(0, _TAIL)], sem_i).wait()
        _e2v_steps(_TAIL, cout, ncg, ypad_hbm, idx_buf, row_bufs,
                   ex_buf, xd_buf, acc_sh, sems, sem_g)

    _drain_all(min(2, _TAIL if _TAIL else _SCH), row_bufs, acc_sh,
               idx_buf, sems)
    _writeout(acc_sh, out_hbm, c, s, _CHUNK, r)


def _sc_e2v(y_pad, xdst, pvpe, cout, r):
    k = pl.kernel(
        functools.partial(_e2v_body, cout, r),
        out_type=jax.ShapeDtypeStruct((_NC * _N, r), jnp.float32),
        mesh=_mesh,
        scratch_types=[
            pltpu.VMEM((_SCH, 2, _CHUNK), jnp.int32),
            pltpu.VMEM((2, _CHUNK, r), jnp.float32),
            pltpu.VMEM((_CHUNK,), jnp.float32),
            pltpu.VMEM((_N,), jnp.float32),
            pltpu.VMEM_SHARED((_N, r), jnp.float32),
            pltpu.SemaphoreType.DMA,
            pltpu.SemaphoreType.DMA,
            pltpu.SemaphoreType.DMA,
            pltpu.SemaphoreType.DMA,
        ],
        compiler_params=_SC_PARAMS,
    )
    return k(y_pad, xdst, pvpe)


def _conv(h, pvpe, w, b, ae, ad, cout):
    r = cout + 16
    xl_pad, xdst = _tc_linear(h, w, b, ad, cout, r)
    yparts = _sc_v2e(xl_pad, pvpe, r)
    y_pad = _tc_edge(yparts, ae, cout, r)
    xoparts = _sc_e2v(y_pad, xdst.reshape(_N), pvpe, cout, r)
    return xoparts


def kernel(X, pair_v, pair_e, W_layers, b_layers, ae_layers, ad_layers,
           W_out, b_out, ae_out, ad_out):
    h = X.astype(jnp.float32)
    pv3 = pair_v.astype(jnp.int32).reshape(_NW, _NCHUNK, 1, _CHUNK)
    pe3 = pair_e.astype(jnp.int32).reshape(_NW, _NCHUNK, 1, _CHUNK)
    pvpe = jnp.concatenate([pv3, pe3], axis=2)
    for l in range(_DEPTH):
        parts = [
            _conv(h, pvpe, W_layers[l, hd], b_layers[l, hd],
                  ae_layers[l, hd], ad_layers[l, hd], _C)
            for hd in range(_HEADS)
        ]
        h = _tc_vertex(parts, _C, _C + 16)
    xop = _conv(h, pvpe, W_out, b_out, ae_out, ad_out, _K)
    return _tc_vertex([xop], _K, _K + 16)
